# Initial kernel scaffold; baseline (speedup 1.0000x reference)
import jax
import jax.numpy as jnp
from jax.experimental import pallas as pl


def kernel(x, W, b):
    raise NotImplementedError("write your pallas kernel here")


# speedup vs baseline: 1.0290x; 1.0290x over previous
import functools

import jax
import jax.numpy as jnp
from jax.experimental import pallas as pl
from jax.experimental.pallas import tpu as pltpu


def _linear_body(x_ref, w_ref, b_ref, o_ref):
    acc = jax.lax.dot_general(
        x_ref[...],
        w_ref[...],
        ((( 1,), (1,)), ((), ())),
        preferred_element_type=jnp.float32,
    )
    o_ref[...] = acc + b_ref[...]


@functools.partial(jax.jit, static_argnames=())
def kernel(x, W, b):
    m, k = x.shape
    n = W.shape[0]
    bm = min(1024, m)
    bn = min(512, n)
    b2 = b.reshape(1, n)
    grid = (m // bm, n // bn)
    return pl.pallas_call(
        _linear_body,
        out_shape=jax.ShapeDtypeStruct((m, n), jnp.float32),
        grid=grid,
        in_specs=[
            pl.BlockSpec((bm, k), lambda i, j: (i, 0)),
            pl.BlockSpec((bn, k), lambda i, j: (j, 0)),
            pl.BlockSpec((1, bn), lambda i, j: (0, j)),
        ],
        out_specs=pl.BlockSpec((bm, bn), lambda i, j: (i, j)),
        compiler_params=pltpu.CompilerParams(
            dimension_semantics=("parallel", "arbitrary"),
        ),
        name="linear_xwt_bias",
    )(x, W, b2)


# speedup vs baseline: 1.0774x; 1.0471x over previous
import functools

import jax
import jax.numpy as jnp
from jax.experimental import pallas as pl
from jax.experimental.pallas import tpu as pltpu


def _linear_body(x_ref, w_ref, b_ref, o_ref):
    acc = jax.lax.dot_general(
        x_ref[...],
        w_ref[...],
        ((( 1,), (1,)), ((), ())),
        preferred_element_type=jnp.float32,
    )
    o_ref[...] = acc + b_ref[...]


@functools.partial(jax.jit, static_argnames=())
def kernel(x, W, b):
    m, k = x.shape
    n = W.shape[0]
    bm = min(1024, m)
    bn = min(512, n)
    b2 = b.reshape(1, n)
    grid = (m // bm, n // bn)
    nj = grid[1]

    def _snake(i, j):
        return jnp.where(i % 2 == 0, j, nj - 1 - j)

    return pl.pallas_call(
        _linear_body,
        out_shape=jax.ShapeDtypeStruct((m, n), jnp.float32),
        grid=grid,
        in_specs=[
            pl.BlockSpec((bm, k), lambda i, j: (i, 0)),
            pl.BlockSpec((bn, k), lambda i, j: (_snake(i, j), 0)),
            pl.BlockSpec((1, bn), lambda i, j: (0, _snake(i, j))),
        ],
        out_specs=pl.BlockSpec((bm, bn), lambda i, j: (i, _snake(i, j))),
        compiler_params=pltpu.CompilerParams(
            dimension_semantics=("parallel", "arbitrary"),
        ),
        name="linear_xwt_bias",
    )(x, W, b2)
